# baseline (speedup 1.0000x reference)
import functools

import jax
import jax.numpy as jnp
from jax import lax
from jax.experimental import pallas as pl
from jax.experimental.pallas import tpu as pltpu
from jax.experimental.pallas import tpu_sc as plsc

_B = 16384
_NINPUT = 784
_NHIDDEN = 1024
_NLATENT = 32
_NEMB = 8192
_NEMBDIM = 32
_COMMIT = 0.25

_NSPLIT = 1
_CH = _B // _NSPLIT
_BT_ENC = 256
_BT_DEC = 512

_NC = 2
_NS = 16
_NW = _NC * _NS
_IDX_CHUNK = 128


_DN_T = (((1,), (1,)), ((), ()))


def _enc_vq_body(x_hbm, w1_ref, b1_ref, w2_ref, b2_ref, w3_ref, b3_ref,
                 e_ref, z_ref, idx_ref, e2_ref, xbuf, xsem):
    i = pl.program_id(0)
    n = pl.num_programs(0)
    slot = lax.rem(i, 2)
    nslot = lax.rem(i + 1, 2)

    @pl.when(i == 0)
    def _():
        e0 = e_ref[...]
        e2_ref[...] = jnp.sum(e0 * e0, axis=1)[None, :]
        pltpu.make_async_copy(x_hbm.at[pl.ds(0, _BT_ENC)],
                              xbuf.at[0], xsem.at[0]).start()

    @pl.when(i + 1 < n)
    def _():
        pltpu.make_async_copy(x_hbm.at[pl.ds((i + 1) * _BT_ENC, _BT_ENC)],
                              xbuf.at[nslot], xsem.at[nslot]).start()

    pltpu.make_async_copy(x_hbm.at[pl.ds(i * _BT_ENC, _BT_ENC)],
                          xbuf.at[slot], xsem.at[slot]).wait()
    h = lax.dot_general(xbuf[slot], w1_ref[...], _DN_T,
                        preferred_element_type=jnp.float32)
    h = jnp.maximum(h + b1_ref[...], 0.0)
    h = lax.dot_general(h, w2_ref[...], _DN_T,
                        preferred_element_type=jnp.float32)
    h = jnp.maximum(h + b2_ref[...], 0.0)
    z = lax.dot_general(h, w3_ref[...], _DN_T,
                        preferred_element_type=jnp.float32) + b3_ref[...]
    z_ref[...] = z
    s = lax.dot_general(z.astype(jnp.bfloat16), e_ref[...].astype(jnp.bfloat16),
                        (((1,), (1,)), ((), ())),
                        preferred_element_type=jnp.float32)
    score = s - e2_ref[...]
    m = jnp.max(score, axis=1, keepdims=True)
    match = (score == m).astype(jnp.float32)
    colsf = lax.broadcasted_iota(jnp.int32, (1, _NEMB), 1).astype(jnp.float32)
    idxf = lax.dot_general(match, colsf, (((1,), (1,)), ((), ())),
                           preferred_element_type=jnp.float32)
    idx_ref[...] = jnp.minimum(idxf, float(_NEMB - 1)).astype(jnp.int32)


def _dec_body(q_ref, z_ref, w4_ref, b4_ref, w5_ref, b5_ref, w6_ref, b6_ref,
              out_hbm, loss_ref, obuf, osem):
    i = pl.program_id(0)
    n = pl.num_programs(0)
    slot = lax.rem(i, 2)
    q = q_ref[...]
    z = z_ref[...]
    qst = z + (q - z)
    h = lax.dot_general(qst, w4_ref[...], _DN_T,
                        preferred_element_type=jnp.float32)
    h = jnp.maximum(h + b4_ref[...], 0.0)
    h = lax.dot_general(h, w5_ref[...], _DN_T,
                        preferred_element_type=jnp.float32)
    h = jnp.maximum(h + b5_ref[...], 0.0)
    o = lax.dot_general(h, w6_ref[...], _DN_T,
                        preferred_element_type=jnp.float32) + b6_ref[...]

    @pl.when(i >= 2)
    def _():
        pltpu.make_async_copy(obuf.at[slot],
                              out_hbm.at[pl.ds((i - 2) * _BT_DEC, _BT_DEC)],
                              osem.at[slot]).wait()

    obuf[slot] = 1.0 / (1.0 + jnp.exp(-o))
    pltpu.make_async_copy(obuf.at[slot],
                          out_hbm.at[pl.ds(i * _BT_DEC, _BT_DEC)],
                          osem.at[slot]).start()
    part = jnp.sum((q - z) ** 2, keepdims=True)[:1, :1]
    loss_ref[...] = jnp.where(i == 0, part, loss_ref[...] + part)

    @pl.when(i == n - 1)
    def _():
        pltpu.make_async_copy(obuf.at[lax.rem(i + 1, 2)],
                              out_hbm.at[pl.ds((i - 1) * _BT_DEC, _BT_DEC)],
                              osem.at[lax.rem(i + 1, 2)]).wait()
        pltpu.make_async_copy(obuf.at[slot],
                              out_hbm.at[pl.ds(i * _BT_DEC, _BT_DEC)],
                              osem.at[slot]).wait()


@functools.cache
def _sc_gather_fn(n_rows):
    bpw = n_rows // _NW
    nchunk = bpw // _IDX_CHUNK
    mesh = plsc.VectorSubcoreMesh(core_axis_name="c", subcore_axis_name="s")

    @functools.partial(
        pl.kernel,
        out_type=jax.ShapeDtypeStruct((n_rows, _NEMBDIM), jnp.float32),
        mesh=mesh,
        scratch_types=[
            pltpu.VMEM((nchunk, _IDX_CHUNK), jnp.int32),
            pltpu.VMEM((bpw, _NEMBDIM), jnp.float32),
            pltpu.VMEM_SHARED((_NEMB, _NEMBDIM), jnp.float32),
            pltpu.SemaphoreType.DMA,
        ],
        compiler_params=pltpu.CompilerParams(use_tc_tiling_on_sc=False),
    )
    def _sc_gather(table_hbm, idx_hbm, out_hbm, idx_v, rows_v, e_sh, sem):
        sid = lax.axis_index("s")
        wid = sid * _NC + lax.axis_index("c")
        ept = _NEMB // _NS
        pltpu.sync_copy(table_hbm.at[pl.ds(sid * ept, ept)],
                        e_sh.at[pl.ds(sid * ept, ept)])
        pltpu.sync_copy(idx_hbm.at[wid], idx_v)
        plsc.subcore_barrier()
        cps = []
        for j in range(nchunk):
            cps.append(pltpu.async_copy(
                e_sh.at[idx_v.at[j]],
                rows_v.at[pl.ds(j * _IDX_CHUNK, _IDX_CHUNK)], sem))
        for cp in cps:
            cp.wait()
        pltpu.sync_copy(rows_v, out_hbm.at[pl.ds(wid * bpw, bpw)])

    return _sc_gather


def _enc_call(xc, w1t, b1r, w2t, b2r, w3t, b3r, e):
    n = xc.shape[0]
    nb = n // _BT_ENC
    return pl.pallas_call(
        _enc_vq_body,
        grid=(nb,),
        in_specs=[
            pl.BlockSpec(memory_space=pl.ANY),
            pl.BlockSpec((_NHIDDEN, _NINPUT), lambda i: (0, 0)),
            pl.BlockSpec((1, _NHIDDEN), lambda i: (0, 0)),
            pl.BlockSpec((_NHIDDEN, _NHIDDEN), lambda i: (0, 0)),
            pl.BlockSpec((1, _NHIDDEN), lambda i: (0, 0)),
            pl.BlockSpec((_NLATENT, _NHIDDEN), lambda i: (0, 0)),
            pl.BlockSpec((1, _NLATENT), lambda i: (0, 0)),
            pl.BlockSpec((_NEMB, _NEMBDIM), lambda i: (0, 0)),
        ],
        out_specs=[
            pl.BlockSpec((_BT_ENC, _NLATENT), lambda i: (i, 0)),
            pl.BlockSpec((_BT_ENC, 1), lambda i: (i, 0)),
        ],
        out_shape=[
            jax.ShapeDtypeStruct((n, _NLATENT), jnp.float32),
            jax.ShapeDtypeStruct((n, 1), jnp.int32),
        ],
        scratch_shapes=[
            pltpu.VMEM((1, _NEMB), jnp.float32),
            pltpu.VMEM((2, _BT_ENC, _NINPUT), jnp.float32),
            pltpu.SemaphoreType.DMA((2,)),
        ],
    )(xc, w1t, b1r, w2t, b2r, w3t, b3r, e)


def _dec_call(qc, zc, w4t, b4r, w5t, b5r, w6t, b6r):
    n = qc.shape[0]
    nb = n // _BT_DEC
    return pl.pallas_call(
        _dec_body,
        grid=(nb,),
        in_specs=[
            pl.BlockSpec((_BT_DEC, _NEMBDIM), lambda i: (i, 0)),
            pl.BlockSpec((_BT_DEC, _NLATENT), lambda i: (i, 0)),
            pl.BlockSpec((_NHIDDEN, _NLATENT), lambda i: (0, 0)),
            pl.BlockSpec((1, _NHIDDEN), lambda i: (0, 0)),
            pl.BlockSpec((_NHIDDEN, _NHIDDEN), lambda i: (0, 0)),
            pl.BlockSpec((1, _NHIDDEN), lambda i: (0, 0)),
            pl.BlockSpec((_NINPUT, _NHIDDEN), lambda i: (0, 0)),
            pl.BlockSpec((1, _NINPUT), lambda i: (0, 0)),
        ],
        out_specs=[
            pl.BlockSpec(memory_space=pl.ANY),
            pl.BlockSpec((1, 1), lambda i: (0, 0)),
        ],
        out_shape=[
            jax.ShapeDtypeStruct((n, _NINPUT), jnp.float32),
            jax.ShapeDtypeStruct((1, 1), jnp.float32),
        ],
        scratch_shapes=[
            pltpu.VMEM((2, _BT_DEC, _NINPUT), jnp.float32),
            pltpu.SemaphoreType.DMA((2,)),
        ],
    )(qc, zc, w4t, b4r, w5t, b5r, w6t, b6r)


def kernel(x, W1, b1, W2, b2, W3, b3, E, W4, b4, W5, b5, W6, b6):
    w1t, w2t, w3t = W1, W2, W3
    w4t, w5t, w6t = W4, W5, W6
    b1r, b2r, b3r = b1[None, :], b2[None, :], b3[None, :]
    b4r, b5r, b6r = b4[None, :], b5[None, :], b6[None, :]
    gather = _sc_gather_fn(_CH)
    nchunk = (_CH // _NW) // _IDX_CHUNK
    outs, lparts = [], []
    for c in range(_NSPLIT):
        xc = x if _NSPLIT == 1 else lax.slice_in_dim(x, c * _CH, (c + 1) * _CH)
        zc, idxc = _enc_call(xc, w1t, b1r, w2t, b2r, w3t, b3r, E)
        qc = gather(E, idxc.reshape(_NW, nchunk, _IDX_CHUNK))
        oc, lc = _dec_call(qc, zc, w4t, b4r, w5t, b5r, w6t, b6r)
        outs.append(oc)
        lparts.append(lc)
    out = outs[0] if _NSPLIT == 1 else jnp.concatenate(outs, axis=0)
    scale = (1.0 + _COMMIT) / (_B * _NLATENT)
    loss = (sum(lp[0, 0] for lp in lparts) * scale).reshape(())
    return (out, loss)


# speedup vs baseline: 1.2978x; 1.2978x over previous
import functools

import jax
import jax.numpy as jnp
from jax import lax
from jax.experimental import pallas as pl
from jax.experimental.pallas import tpu as pltpu
from jax.experimental.pallas import tpu_sc as plsc

_B = 16384
_NINPUT = 784
_NHIDDEN = 1024
_NLATENT = 32
_NEMB = 8192
_NEMBDIM = 32
_COMMIT = 0.25

_BT_ENC = 256
_BT_DEC = 512

_NC = 2
_NS = 16
_NW = _NC * _NS
_IDX_CHUNK = 128

_DN_NN = (((1,), (0,)), ((), ()))
_DN_TN = (((0,), (0,)), ((), ()))


def _enc_vq_body(xt_ref, w1t_ref, b1_ref, w2t_ref, b2_ref, w3t_ref, b3_ref,
                 et_ref, z_ref, idx_ref, e2_ref):
    @pl.when(pl.program_id(0) == 0)
    def _():
        e0 = et_ref[...]
        e2_ref[...] = jnp.sum(e0 * e0, axis=0, keepdims=True)

    h = lax.dot_general(xt_ref[...], w1t_ref[...], _DN_TN,
                        preferred_element_type=jnp.float32)
    h = jnp.maximum(h + b1_ref[...], 0.0)
    h = lax.dot_general(h, w2t_ref[...], _DN_NN,
                        preferred_element_type=jnp.float32)
    h = jnp.maximum(h + b2_ref[...], 0.0)
    z = lax.dot_general(h, w3t_ref[...], _DN_NN,
                        preferred_element_type=jnp.float32) + b3_ref[...]
    z_ref[...] = z
    s = lax.dot_general(z.astype(jnp.bfloat16), et_ref[...].astype(jnp.bfloat16),
                        _DN_NN, preferred_element_type=jnp.float32)
    score = s - e2_ref[...]
    m = jnp.max(score, axis=1, keepdims=True)
    match = (score == m).astype(jnp.float32)
    colsf = lax.broadcasted_iota(jnp.int32, (1, _NEMB), 1).astype(jnp.float32)
    idxf = lax.dot_general(match, colsf, (((1,), (1,)), ((), ())),
                           preferred_element_type=jnp.float32)
    idx_ref[...] = jnp.minimum(idxf, float(_NEMB - 1)).astype(jnp.int32)


def _dec_body(q_ref, z_ref, w4t_ref, b4_ref, w5t_ref, b5_ref, w6t_ref, b6t_ref,
              outt_ref, loss_ref):
    q = q_ref[...]
    z = z_ref[...]
    qst = z + (q - z)
    h = lax.dot_general(qst, w4t_ref[...], _DN_NN,
                        preferred_element_type=jnp.float32)
    h = jnp.maximum(h + b4_ref[...], 0.0)
    h = lax.dot_general(h, w5t_ref[...], _DN_NN,
                        preferred_element_type=jnp.float32)
    h = jnp.maximum(h + b5_ref[...], 0.0)
    ot = lax.dot_general(w6t_ref[...], h, (((0,), (1,)), ((), ())),
                         preferred_element_type=jnp.float32)
    ot = ot + b6t_ref[...]
    outt_ref[...] = 1.0 / (1.0 + jnp.exp(-ot))
    part = jnp.sum((q - z) ** 2, keepdims=True)[:1, :1]
    i = pl.program_id(0)
    loss_ref[...] = jnp.where(i == 0, part, loss_ref[...] + part)


@functools.cache
def _sc_gather_fn(n_rows):
    bpw = n_rows // _NW
    nchunk = bpw // _IDX_CHUNK
    mesh = plsc.VectorSubcoreMesh(core_axis_name="c", subcore_axis_name="s")

    @functools.partial(
        pl.kernel,
        out_type=jax.ShapeDtypeStruct((n_rows, _NEMBDIM), jnp.float32),
        mesh=mesh,
        scratch_types=[
            pltpu.VMEM((nchunk, _IDX_CHUNK), jnp.int32),
            pltpu.VMEM((bpw, _NEMBDIM), jnp.float32),
            pltpu.VMEM_SHARED((_NEMB, _NEMBDIM), jnp.float32),
            pltpu.SemaphoreType.DMA,
        ],
        compiler_params=pltpu.CompilerParams(use_tc_tiling_on_sc=False),
    )
    def _sc_gather(table_hbm, idx_hbm, out_hbm, idx_v, rows_v, e_sh, sem):
        sid = lax.axis_index("s")
        wid = sid * _NC + lax.axis_index("c")
        ept = _NEMB // _NS
        pltpu.sync_copy(table_hbm.at[pl.ds(sid * ept, ept)],
                        e_sh.at[pl.ds(sid * ept, ept)])
        pltpu.sync_copy(idx_hbm.at[wid], idx_v)
        plsc.subcore_barrier()
        cps = []
        for j in range(nchunk):
            cps.append(pltpu.async_copy(
                e_sh.at[idx_v.at[j]],
                rows_v.at[pl.ds(j * _IDX_CHUNK, _IDX_CHUNK)], sem))
        for cp in cps:
            cp.wait()
        pltpu.sync_copy(rows_v, out_hbm.at[pl.ds(wid * bpw, bpw)])

    return _sc_gather


def _enc_call(xt, w1t, b1r, w2t, b2r, w3t, b3r, et):
    n = xt.shape[1]
    nb = n // _BT_ENC
    return pl.pallas_call(
        _enc_vq_body,
        grid=(nb,),
        in_specs=[
            pl.BlockSpec((_NINPUT, _BT_ENC), lambda i: (0, i)),
            pl.BlockSpec((_NINPUT, _NHIDDEN), lambda i: (0, 0)),
            pl.BlockSpec((1, _NHIDDEN), lambda i: (0, 0)),
            pl.BlockSpec((_NHIDDEN, _NHIDDEN), lambda i: (0, 0)),
            pl.BlockSpec((1, _NHIDDEN), lambda i: (0, 0)),
            pl.BlockSpec((_NHIDDEN, _NLATENT), lambda i: (0, 0)),
            pl.BlockSpec((1, _NLATENT), lambda i: (0, 0)),
            pl.BlockSpec((_NEMBDIM, _NEMB), lambda i: (0, 0)),
        ],
        out_specs=[
            pl.BlockSpec((_BT_ENC, _NLATENT), lambda i: (i, 0)),
            pl.BlockSpec((_BT_ENC, 1), lambda i: (i, 0)),
        ],
        out_shape=[
            jax.ShapeDtypeStruct((n, _NLATENT), jnp.float32),
            jax.ShapeDtypeStruct((n, 1), jnp.int32),
        ],
        scratch_shapes=[pltpu.VMEM((1, _NEMB), jnp.float32)],
    )(xt, w1t, b1r, w2t, b2r, w3t, b3r, et)


def _dec_call(qc, zc, w4t, b4r, w5t, b5r, w6, b6c):
    n = qc.shape[0]
    nb = n // _BT_DEC
    return pl.pallas_call(
        _dec_body,
        grid=(nb,),
        in_specs=[
            pl.BlockSpec((_BT_DEC, _NEMBDIM), lambda i: (i, 0)),
            pl.BlockSpec((_BT_DEC, _NLATENT), lambda i: (i, 0)),
            pl.BlockSpec((_NLATENT, _NHIDDEN), lambda i: (0, 0)),
            pl.BlockSpec((1, _NHIDDEN), lambda i: (0, 0)),
            pl.BlockSpec((_NHIDDEN, _NHIDDEN), lambda i: (0, 0)),
            pl.BlockSpec((1, _NHIDDEN), lambda i: (0, 0)),
            pl.BlockSpec((_NHIDDEN, _NINPUT), lambda i: (0, 0)),
            pl.BlockSpec((_NINPUT, 1), lambda i: (0, 0)),
        ],
        out_specs=[
            pl.BlockSpec((_NINPUT, _BT_DEC), lambda i: (0, i)),
            pl.BlockSpec((1, 1), lambda i: (0, 0)),
        ],
        out_shape=[
            jax.ShapeDtypeStruct((_NINPUT, n), jnp.float32),
            jax.ShapeDtypeStruct((1, 1), jnp.float32),
        ],
    )(qc, zc, w4t, b4r, w5t, b5r, w6, b6c)


def kernel(x, W1, b1, W2, b2, W3, b3, E, W4, b4, W5, b5, W6, b6):
    xt = x.T
    b1r, b2r, b3r = b1[None, :], b2[None, :], b3[None, :]
    b4r, b5r = b4[None, :], b5[None, :]
    gather = _sc_gather_fn(_B)
    nchunk = (_B // _NW) // _IDX_CHUNK

    zc, idxc = _enc_call(xt, W1.T, b1r, W2.T, b2r, W3.T, b3r, E.T)
    qc = gather(E, idxc.reshape(_NW, nchunk, _IDX_CHUNK))
    outt, lsum = _dec_call(qc, zc, W4.T, b4r, W5.T, b5r, W6.T, b6[:, None])
    scale = (1.0 + _COMMIT) / (_B * _NLATENT)
    loss = (lsum[0, 0] * scale).reshape(())
    return (outt.T, loss)
